# baseline (speedup 1.0000x reference)
import functools

import jax
import jax.numpy as jnp
from jax import lax
from jax.experimental import pallas as pl
from jax.experimental.pallas import tpu as pltpu
from jax.experimental.pallas import tpu_sc as plsc

_ROWS = 256

_INFO = plsc.get_sparse_core_info()
_NC = _INFO.num_cores
_NS = _INFO.num_subcores
_NW = _NC * _NS

_CHUNK = 16
_S_SC = 1024


def _tc_body(batch, n_chunks, ring, w_hbm, o_hbm, *rest):
    bufs = rest[:ring]
    rsems = rest[ring:2 * ring]
    wsems = rest[2 * ring:3 * ring]

    def read(k):
        pltpu.make_async_copy(
            w_hbm.at[pl.ds(k * _ROWS, _ROWS), :], bufs[k % ring], rsems[k % ring]
        ).start()

    def write_start(k):
        for b in range(batch):
            pltpu.make_async_copy(
                bufs[k % ring], o_hbm.at[b, pl.ds(k * _ROWS, _ROWS), :],
                wsems[k % ring],
            ).start()

    def write_wait(k):
        for b in range(batch):
            pltpu.make_async_copy(
                bufs[k % ring], o_hbm.at[b, pl.ds(k * _ROWS, _ROWS), :],
                wsems[k % ring],
            ).wait()

    for k in range(min(ring, n_chunks)):
        read(k)
    for k in range(n_chunks):
        pltpu.make_async_copy(
            w_hbm.at[pl.ds(k * _ROWS, _ROWS), :], bufs[k % ring], rsems[k % ring]
        ).wait()
        write_start(k)
        p = k - 1
        if p >= 0 and p + ring < n_chunks:
            write_wait(p)
            read(p + ring)
    for p in range(max(0, n_chunks - ring - 1), n_chunks):
        if p + ring >= n_chunks:
            write_wait(p)


def _tc_call(W_pos, batch, s_tc, d):
    n_chunks = s_tc // _ROWS
    ring = n_chunks
    return pl.pallas_call(
        functools.partial(_tc_body, batch, n_chunks, ring),
        in_specs=[pl.BlockSpec(memory_space=pl.ANY)],
        out_specs=pl.BlockSpec(memory_space=pl.ANY),
        out_shape=jax.ShapeDtypeStruct((batch, s_tc, d), jnp.float32),
        scratch_shapes=(
            [pltpu.VMEM((_ROWS, d), jnp.float32)] * ring
            + [pltpu.SemaphoreType.DMA] * (2 * ring)
        ),
    )(W_pos)


def _sc_body(n_chunks, batch, base_row, w_hbm, out_hbm, buf0, buf1, sem0, sem1,
             wsem):
    wid = lax.axis_index("s") * _NC + lax.axis_index("c")
    base = base_row + wid * (n_chunks * _CHUNK)
    obase = wid * (n_chunks * _CHUNK)

    bufs = (buf0, buf1)
    in_sems = (sem0, sem1)

    def gather(k):
        pltpu.async_copy(
            w_hbm.at[pl.ds(base + k * _CHUNK, _CHUNK), :],
            bufs[k % 2], in_sems[k % 2],
        )

    gather(0)
    for k in range(n_chunks):
        slot = k % 2
        pltpu.make_async_copy(
            w_hbm.at[pl.ds(base + k * _CHUNK, _CHUNK), :],
            bufs[slot], in_sems[slot],
        ).wait()
        if k + 1 < n_chunks:
            gather(k + 1)
        r = obase + k * _CHUNK
        for b in range(batch):
            pltpu.async_copy(bufs[slot], out_hbm.at[b, pl.ds(r, _CHUNK), :], wsem)
        for b in range(batch):
            pltpu.make_async_copy(
                bufs[slot], out_hbm.at[b, pl.ds(r, _CHUNK), :], wsem
            ).wait()


def _sc_call(W_pos, batch, s_tc, s_sc, d):
    n_chunks = s_sc // (_NW * _CHUNK)
    mesh = plsc.VectorSubcoreMesh(core_axis_name="c", subcore_axis_name="s")
    k = functools.partial(
        pl.kernel,
        mesh=mesh,
        out_type=jax.ShapeDtypeStruct((batch, s_sc, d), jnp.float32),
        scratch_types=[
            pltpu.VMEM((_CHUNK, d), jnp.float32),
            pltpu.VMEM((_CHUNK, d), jnp.float32),
            pltpu.SemaphoreType.DMA,
            pltpu.SemaphoreType.DMA,
            pltpu.SemaphoreType.DMA,
        ],
    )(functools.partial(_sc_body, n_chunks, batch, s_tc))
    return k(W_pos)


def kernel(tokens, W_pos):
    B, S = tokens.shape
    D = W_pos.shape[1]
    s_tc = S - _S_SC
    tc_out = _tc_call(W_pos, B, s_tc, D)
    sc_out = _sc_call(W_pos, B, s_tc, _S_SC, D)
    return jnp.concatenate([tc_out, sc_out], axis=1)


# speedup vs baseline: 3.0138x; 3.0138x over previous
import functools

import jax
import jax.numpy as jnp
from jax.experimental import pallas as pl
from jax.experimental.pallas import tpu as pltpu

_ROWS = 256
_RING = 16


def _dma_body(batch, n_chunks, w_hbm, o_hbm, *rest):
    bufs = rest[:_RING]
    rsems = rest[_RING:2 * _RING]
    wsems = rest[2 * _RING:3 * _RING]

    def read(k):
        pltpu.make_async_copy(
            w_hbm.at[pl.ds(k * _ROWS, _ROWS), :], bufs[k % _RING], rsems[k % _RING]
        ).start()

    def write_start(k):
        for b in range(batch):
            pltpu.make_async_copy(
                bufs[k % _RING], o_hbm.at[b, pl.ds(k * _ROWS, _ROWS), :],
                wsems[k % _RING],
            ).start()

    def write_wait(k):
        for b in range(batch):
            pltpu.make_async_copy(
                bufs[k % _RING], o_hbm.at[b, pl.ds(k * _ROWS, _ROWS), :],
                wsems[k % _RING],
            ).wait()

    for k in range(_RING):
        read(k)
    for k in range(n_chunks):
        pltpu.make_async_copy(
            w_hbm.at[pl.ds(k * _ROWS, _ROWS), :], bufs[k % _RING], rsems[k % _RING]
        ).wait()
        write_start(k)
        p = k - 1
        if p >= 0 and p + _RING < n_chunks:
            write_wait(p)
            read(p + _RING)
    for p in range(max(0, n_chunks - _RING - 1), n_chunks):
        if p + _RING >= n_chunks:
            write_wait(p)


def kernel(tokens, W_pos):
    B, S = tokens.shape
    D = W_pos.shape[1]
    n_chunks = S // _ROWS

    return pl.pallas_call(
        functools.partial(_dma_body, B, n_chunks),
        in_specs=[pl.BlockSpec(memory_space=pl.ANY)],
        out_specs=pl.BlockSpec(memory_space=pl.ANY),
        out_shape=jax.ShapeDtypeStruct((B, S, D), jnp.float32),
        scratch_shapes=(
            [pltpu.VMEM((_ROWS, D), jnp.float32)] * _RING
            + [pltpu.SemaphoreType.DMA] * (2 * _RING)
        ),
    )(W_pos)
